# baseline (speedup 1.0000x reference)
import functools

import jax
import jax.numpy as jnp
from jax import lax
from jax.experimental import pallas as pl
from jax.experimental.pallas import tpu as pltpu
from jax.experimental.pallas import tpu_sc as plsc

GAMMA = 1.0

NC = 2
NS = 16
NW = NC * NS
LANES = 16


def _sc_body(nsteps, g, chunk, col_stride, p_valid, d,
             idx_hbm, feat_hbm, out_hbm, idxl_v, idxr_v, idxfl_v, idxfr_v,
             la, ra, fla, fra, lb, rb, flb, frb, out_stage, sem_a, sem_b):
  wid = lax.axis_index("s") * NC + lax.axis_index("c")
  base = wid * chunk

  idx_bufs = (idxl_v, idxr_v, idxfl_v, idxfr_v)
  for c in range(4):
    pltpu.sync_copy(idx_hbm.at[pl.ds(c * col_stride + base, chunk + g)],
                    idx_bufs[c])

  buf_a = (la, ra, fla, fra)
  buf_b = (lb, rb, flb, frb)
  nq = d // LANES
  lane = lax.iota(jnp.int32, LANES)
  rots = [(lane + s) & (LANES - 1) for s in (8, 4, 2, 1)]
  dnums = lax.GatherDimensionNumbers(
      offset_dims=(), collapsed_slice_dims=(0,), start_index_map=(0,))

  def hsum(x):
    for perm in rots:
      rot = lax.gather(x, perm[:, None], dnums, slice_sizes=(1,),
                       mode=lax.GatherScatterMode.PROMISE_IN_BOUNDS)
      x = x + rot
    return x

  def gather_step(t, bufs, sem):
    for c in range(4):
      pltpu.async_copy(feat_hbm.at[idx_bufs[c].at[pl.ds(t * g, g)]],
                       bufs[c], sem)

  def wait_step(t, bufs, sem):
    for c in range(4):
      pltpu.make_async_copy(feat_hbm.at[idx_bufs[c].at[pl.ds(t * g, g)]],
                            bufs[c], sem).wait()

  unroll = 4

  def compute_step(t, bufs, acc):
    rows_l, rows_r, rows_fl, rows_fr = bufs

    def pair_group(pg, acc2):
      for k in range(unroll):
        i = pg * unroll + k
        u = None
        v = None
        for q in range(nq):
          sl = pl.ds(q * LANES, LANES)
          lv = rows_l[i, sl]
          rv = rows_r[i, sl]
          flv = rows_fl[i, sl]
          frv = rows_fr[i, sl]
          a = jnp.abs(lv - rv)
          du = a - jnp.abs(lv - frv)
          dv = a - jnp.abs(flv - rv)
          u = du if u is None else u + du
          v = dv if v is None else v + dv
        su = hsum(u)
        sv = hsum(v)
        contrib = (jnp.maximum(GAMMA + su, 0.0) +
                   jnp.maximum(GAMMA + sv, 0.0))
        valid = (base + t * g + i) < p_valid
        acc2 = acc2 + jnp.where(valid, contrib, 0.0)
      return acc2

    return lax.fori_loop(0, g // unroll, pair_group, acc)

  gather_step(0, buf_a, sem_a)

  def body2(m, acc):
    t0 = 2 * m
    gather_step(t0 + 1, buf_b, sem_b)
    wait_step(t0, buf_a, sem_a)
    acc = compute_step(t0, buf_a, acc)
    gather_step(t0 + 2, buf_a, sem_a)
    wait_step(t0 + 1, buf_b, sem_b)
    return compute_step(t0 + 1, buf_b, acc)

  acc = lax.fori_loop(0, nsteps // 2, body2,
                      jnp.zeros((LANES,), jnp.float32))
  wait_step(nsteps, buf_a, sem_a)

  out_stage[...] = jnp.where(lane == 0, acc, 0.0)
  pltpu.sync_copy(out_stage, out_hbm.at[wid])


def _build_sc_call(p_valid, v_rows, d, g):
  chunk = ((p_valid + 2 * NW * g - 1) // (2 * NW * g)) * 2 * g
  nsteps = chunk // g
  col_stride = NW * chunk + g
  mesh = plsc.VectorSubcoreMesh(core_axis_name="c", subcore_axis_name="s")
  body = functools.partial(_sc_body, nsteps, g, chunk, col_stride, p_valid, d)
  row_buf = pltpu.VMEM((g, d), jnp.float32)
  idx_buf = pltpu.VMEM((chunk + g,), jnp.int32)
  return chunk, col_stride, pl.kernel(
      body,
      out_type=jax.ShapeDtypeStruct((NW, LANES), jnp.float32),
      mesh=mesh,
      scratch_types=[
          idx_buf, idx_buf, idx_buf, idx_buf,
          row_buf, row_buf, row_buf, row_buf,
          row_buf, row_buf, row_buf, row_buf,
          pltpu.VMEM((LANES,), jnp.float32),
          pltpu.SemaphoreType.DMA,
          pltpu.SemaphoreType.DMA,
      ],
  )


def kernel(pairs, features):
  p, _ = pairs.shape
  v_rows, d = features.shape
  g = 64
  chunk, col_stride, call = _build_sc_call(p, v_rows, d, g)
  idx = jnp.zeros((4, col_stride), jnp.int32).at[:, :p].set(pairs.T)
  partials = call(idx.reshape(-1), features)
  return jnp.sum(partials) / v_rows
